# baseline (speedup 1.0000x reference)
import functools

import jax
import jax.numpy as jnp
from jax import lax
from jax.experimental import pallas as pl
from jax.experimental.pallas import tpu as pltpu
from jax.experimental.pallas import tpu_sc as plsc

N = 10000
E = 320000
D = 128
H = 128
C = 40
CP = 64
NP = 10240

NC = 2
NS = 16
NW = NC * NS
CHUNK = 128
RPL = 16
ROWS_TOTAL = 2560
EP = ROWS_TOTAL * CHUNK
ROWS_PW = ROWS_TOTAL // NW
NSLICE = NP // NS
ZCOPIES = NSLICE // CHUNK


def _fill_rows(ref, rows, width, value):
    @pl.loop(0, rows)
    def _(r):
        @pl.loop(0, width // 16)
        def _(c):
            ref.at[pl.ds(r, 1), pl.ds(c * 16, 16)][...] = jnp.full(
                (1, 16), value, jnp.float32
            )


def _deg_body(dst_hbm, degp_hbm, idxv, onesv, zerov, deg_sh, sem):
    cid = lax.axis_index("c")
    sid = lax.axis_index("s")
    wid = sid * NC + cid

    _fill_rows(onesv, CHUNK, 16, 1.0)
    _fill_rows(zerov, CHUNK, 16, 0.0)

    @pl.loop(0, ZCOPIES)
    def _(k):
        pltpu.sync_copy(zerov, deg_sh.at[pl.ds(sid * NSLICE + k * CHUNK, CHUNK)])

    plsc.subcore_barrier()

    @pl.loop(0, ROWS_PW, step=RPL)
    def _(r0):
        pltpu.sync_copy(dst_hbm.at[pl.ds(wid * ROWS_PW + r0, RPL)], idxv)

        @pl.loop(0, RPL)
        def _(j):
            pltpu.sync_copy(onesv, deg_sh.at[idxv.at[j]], add=True)

    plsc.subcore_barrier()

    @pl.loop(0, ZCOPIES)
    def _(k):
        off = sid * NSLICE + k * CHUNK
        pltpu.sync_copy(
            deg_sh.at[pl.ds(off, CHUNK)], degp_hbm.at[pl.ds(cid * NP + off, CHUNK)]
        )


@functools.cache
def _deg_kernel():
    return functools.partial(
        pl.kernel,
        out_type=jax.ShapeDtypeStruct((NC * NP, 16), jnp.float32),
        mesh=plsc.VectorSubcoreMesh(core_axis_name="c", subcore_axis_name="s"),
        compiler_params=pltpu.CompilerParams(use_tc_tiling_on_sc=False),
        scratch_types=[
            pltpu.VMEM((RPL, CHUNK), jnp.int32),
            pltpu.VMEM((CHUNK, 16), jnp.float32),
            pltpu.VMEM((CHUNK, 16), jnp.float32),
            pltpu.VMEM_SHARED((NP, 16), jnp.float32),
            pltpu.SemaphoreType.DMA,
        ],
    )(_deg_body)


RPT = ROWS_TOTAL // NS


def _agg_body(W, NB, NSTAGE, h_hbm, src_hbm, dst_hbm, outp_hbm, srcv, dstv, *rest):
    bufs = rest[:NB]
    tab_sh = rest[NB]
    acc_sh = rest[NB + 1]
    gsems = rest[NB + 2 :]
    cid = lax.axis_index("c")
    sid = lax.axis_index("s")

    pltpu.sync_copy(
        h_hbm.at[pl.ds(cid * NP + sid * NSLICE, NSLICE)],
        tab_sh.at[pl.ds(sid * NSLICE, NSLICE)],
    )

    _fill_rows(bufs[0], CHUNK, W, 0.0)

    @pl.loop(0, ZCOPIES)
    def _(k):
        pltpu.sync_copy(bufs[0], acc_sh.at[pl.ds(sid * NSLICE + k * CHUNK, CHUNK)])

    plsc.subcore_barrier()

    for h0 in range(0, RPT, NSTAGE):
        base = sid * RPT + h0
        pltpu.sync_copy(src_hbm.at[pl.ds(base, NSTAGE)], srcv)
        pltpu.sync_copy(dst_hbm.at[pl.ds(base, NSTAGE)], dstv)

        for b in range(NB):
            pltpu.async_copy(tab_sh.at[srcv.at[b]], bufs[b], gsems[b])

        @pl.loop(0, NSTAGE - NB, step=NB)
        def _(j):
            for b in range(NB):
                pltpu.make_async_copy(
                    tab_sh.at[srcv.at[j + b]], bufs[b], gsems[b]
                ).wait()
                pltpu.sync_copy(bufs[b], acc_sh.at[dstv.at[j + b]], add=True)
                pltpu.async_copy(tab_sh.at[srcv.at[j + NB + b]], bufs[b], gsems[b])

        for b in range(NB):
            jl = NSTAGE - NB + b
            pltpu.make_async_copy(tab_sh.at[srcv.at[jl]], bufs[b], gsems[b]).wait()
            pltpu.sync_copy(bufs[b], acc_sh.at[dstv.at[jl]], add=True)

    plsc.subcore_barrier()

    @pl.loop(0, ZCOPIES)
    def _(k):
        off = sid * NSLICE + k * CHUNK
        pltpu.sync_copy(
            acc_sh.at[pl.ds(off, CHUNK)], outp_hbm.at[pl.ds(cid * NP + off, CHUNK)]
        )


@functools.cache
def _make_agg(W, NB, NSTAGE):
    return functools.partial(
        pl.kernel,
        out_type=jax.ShapeDtypeStruct((NC * NP, W), jnp.float32),
        mesh=plsc.VectorSubcoreMesh(core_axis_name="c", subcore_axis_name="s"),
        compiler_params=pltpu.CompilerParams(use_tc_tiling_on_sc=False),
        scratch_types=[
            pltpu.VMEM((NSTAGE, CHUNK), jnp.int32),
            pltpu.VMEM((NSTAGE, CHUNK), jnp.int32),
            *[pltpu.VMEM((CHUNK, W), jnp.float32) for _ in range(NB)],
            pltpu.VMEM_SHARED((NP, W), jnp.float32),
            pltpu.VMEM_SHARED((NP, W), jnp.float32),
            *[pltpu.SemaphoreType.DMA for _ in range(NB)],
        ],
    )(functools.partial(_agg_body, W, NB, NSTAGE))


def _tc1_body(degp_ref, x_ref, w1_ref, h1p_ref, dis_ref):
    dp = degp_ref[...]
    deg = (dp[:NP] + dp[NP:]).sum(axis=1, keepdims=True) * (1.0 / 16.0) + 1.0
    dis = lax.rsqrt(deg)
    h = jnp.dot(
        x_ref[...],
        w1_ref[...],
        preferred_element_type=jnp.float32,
        precision=lax.Precision.DEFAULT,
    )
    hp = h * dis[:N]
    zpad = jnp.zeros((NP - N, H // 2), jnp.float32)
    h1p_ref[pl.ds(0, N)] = hp[:, : H // 2]
    h1p_ref[pl.ds(N, NP - N)] = zpad
    h1p_ref[pl.ds(NP, N)] = hp[:, H // 2 :]
    h1p_ref[pl.ds(NP + N, NP - N)] = zpad
    dis_ref[...] = dis


def _tc2_body(acc_ref, h1p_ref, dis_ref, b1_ref, w2p_ref, gp_ref):
    dis = dis_ref[...]
    b1 = b1_ref[...]
    hw = H // 2
    hr_lo = jnp.maximum((acc_ref[:NP] + h1p_ref[:NP]) * dis + b1[None, :hw], 0.0)
    hr_hi = jnp.maximum((acc_ref[NP:] + h1p_ref[NP:]) * dis + b1[None, hw:], 0.0)
    g = jnp.dot(
        hr_lo,
        w2p_ref[:hw],
        preferred_element_type=jnp.float32,
        precision=lax.Precision.DEFAULT,
    ) + jnp.dot(
        hr_hi,
        w2p_ref[hw:],
        preferred_element_type=jnp.float32,
        precision=lax.Precision.DEFAULT,
    )
    gd = g * dis
    gp_ref[:NP] = gd[:, : CP // 2]
    gp_ref[NP:] = gd[:, CP // 2 :]


def _tc3_body(acc_ref, gp_ref, dis_ref, b2p_ref, feat_ref, logp_ref):
    s = jnp.concatenate(
        [
            acc_ref[pl.ds(0, N)] + gp_ref[pl.ds(0, N)],
            acc_ref[pl.ds(NP, N)] + gp_ref[pl.ds(NP, N)],
        ],
        axis=1,
    )
    out = s * dis_ref[pl.ds(0, N)] + b2p_ref[...][None, :]
    col = lax.broadcasted_iota(jnp.int32, (N, CP), 1)
    valid = col < C
    xm = jnp.where(valid, out, -jnp.inf)
    m = jnp.max(xm, axis=1, keepdims=True)
    e = jnp.where(valid, jnp.exp(out - m), 0.0)
    lse = m + jnp.log(jnp.sum(e, axis=1, keepdims=True))
    feat_ref[...] = out[:, :C]
    logp_ref[...] = (out - lse)[:, :C]


_tc1 = pl.pallas_call(
    _tc1_body,
    out_shape=[
        jax.ShapeDtypeStruct((NC * NP, H // 2), jnp.float32),
        jax.ShapeDtypeStruct((NP, 1), jnp.float32),
    ],
)

_tc2 = pl.pallas_call(
    _tc2_body,
    out_shape=jax.ShapeDtypeStruct((NC * NP, CP // 2), jnp.float32),
)

_tc3 = pl.pallas_call(
    _tc3_body,
    out_shape=[
        jax.ShapeDtypeStruct((N, C), jnp.float32),
        jax.ShapeDtypeStruct((N, C), jnp.float32),
    ],
)


def kernel(x, edge_index, W1, b1, W2, b2):
    src2 = jnp.full((EP,), N, jnp.int32).at[:E].set(edge_index[0]).reshape(
        ROWS_TOTAL, CHUNK
    )
    dst2 = jnp.full((EP,), N, jnp.int32).at[:E].set(edge_index[1]).reshape(
        ROWS_TOTAL, CHUNK
    )
    w2p = jnp.pad(W2, ((0, 0), (0, CP - C)))
    b2p = jnp.pad(b2, (0, CP - C))

    degp = _deg_kernel()(dst2)
    h1p, dis = _tc1(degp, x, W1)
    acc1 = _make_agg(H // 2, 5, 20)(h1p, src2, dst2)
    gp = _tc2(acc1, h1p, dis, b1, w2p)
    acc2 = _make_agg(CP // 2, 5, 20)(gp, src2, dst2)
    feat, logp = _tc3(acc2, gp, dis, b2p)
    return logp, feat


# speedup vs baseline: 1.0652x; 1.0652x over previous
import functools

import jax
import jax.numpy as jnp
from jax import lax
from jax.experimental import pallas as pl
from jax.experimental.pallas import tpu as pltpu
from jax.experimental.pallas import tpu_sc as plsc

N = 10000
E = 320000
D = 128
H = 128
C = 40
CP = 64
NP = 10240

NC = 2
NS = 16
NW = NC * NS
CHUNK = 128
RPL = 16
ROWS_TOTAL = 2560
EP = ROWS_TOTAL * CHUNK
ROWS_PW = ROWS_TOTAL // NW
NSLICE = NP // NS
ZCOPIES = NSLICE // CHUNK


def _fill_rows(ref, rows, width, value):
    @pl.loop(0, rows)
    def _(r):
        @pl.loop(0, width // 16)
        def _(c):
            ref.at[pl.ds(r, 1), pl.ds(c * 16, 16)][...] = jnp.full(
                (1, 16), value, jnp.float32
            )


def _deg_body(dst_hbm, degp_hbm, idxv, onesv, zerov, deg_sh, sem):
    cid = lax.axis_index("c")
    sid = lax.axis_index("s")
    wid = sid * NC + cid

    _fill_rows(onesv, CHUNK, 16, 1.0)
    _fill_rows(zerov, CHUNK, 16, 0.0)

    @pl.loop(0, ZCOPIES)
    def _(k):
        pltpu.sync_copy(zerov, deg_sh.at[pl.ds(sid * NSLICE + k * CHUNK, CHUNK)])

    plsc.subcore_barrier()

    @pl.loop(0, ROWS_PW, step=RPL)
    def _(r0):
        pltpu.sync_copy(dst_hbm.at[pl.ds(wid * ROWS_PW + r0, RPL)], idxv)

        @pl.loop(0, RPL)
        def _(j):
            pltpu.sync_copy(onesv, deg_sh.at[idxv.at[j]], add=True)

    plsc.subcore_barrier()

    @pl.loop(0, ZCOPIES)
    def _(k):
        off = sid * NSLICE + k * CHUNK
        pltpu.sync_copy(
            deg_sh.at[pl.ds(off, CHUNK)], degp_hbm.at[pl.ds(cid * NP + off, CHUNK)]
        )


@functools.cache
def _deg_kernel():
    return functools.partial(
        pl.kernel,
        out_type=jax.ShapeDtypeStruct((NC * NP, 16), jnp.float32),
        mesh=plsc.VectorSubcoreMesh(core_axis_name="c", subcore_axis_name="s"),
        compiler_params=pltpu.CompilerParams(use_tc_tiling_on_sc=False),
        scratch_types=[
            pltpu.VMEM((RPL, CHUNK), jnp.int32),
            pltpu.VMEM((CHUNK, 16), jnp.float32),
            pltpu.VMEM((CHUNK, 16), jnp.float32),
            pltpu.VMEM_SHARED((NP, 16), jnp.float32),
            pltpu.SemaphoreType.DMA,
        ],
    )(_deg_body)


RPT = ROWS_TOTAL // NS


def _agg_body(W, NB, NSTAGE, h_hbm, src_hbm, dst_hbm, outp_hbm, srcv, dstv, *rest):
    bufs = rest[:NB]
    tab_sh = rest[NB]
    acc_sh = rest[NB + 1]
    gsems = rest[NB + 2 :]
    cid = lax.axis_index("c")
    sid = lax.axis_index("s")

    pltpu.sync_copy(
        h_hbm.at[pl.ds(cid * NP + sid * NSLICE, NSLICE)],
        tab_sh.at[pl.ds(sid * NSLICE, NSLICE)],
    )

    _fill_rows(bufs[0], CHUNK, W, 0.0)

    @pl.loop(0, ZCOPIES)
    def _(k):
        pltpu.sync_copy(bufs[0], acc_sh.at[pl.ds(sid * NSLICE + k * CHUNK, CHUNK)])

    plsc.subcore_barrier()

    for h0 in range(0, RPT, NSTAGE):
        base = sid * RPT + h0
        pltpu.sync_copy(src_hbm.at[pl.ds(base, NSTAGE)], srcv)
        pltpu.sync_copy(dst_hbm.at[pl.ds(base, NSTAGE)], dstv)

        for b in range(NB):
            pltpu.async_copy(tab_sh.at[srcv.at[b]], bufs[b], gsems[b])

        @pl.loop(0, NSTAGE - NB, step=NB)
        def _(j):
            for b in range(NB):
                pltpu.make_async_copy(
                    tab_sh.at[srcv.at[j + b]], bufs[b], gsems[b]
                ).wait()
                pltpu.sync_copy(bufs[b], acc_sh.at[dstv.at[j + b]], add=True)
                pltpu.async_copy(tab_sh.at[srcv.at[j + NB + b]], bufs[b], gsems[b])

        for b in range(NB):
            jl = NSTAGE - NB + b
            pltpu.make_async_copy(tab_sh.at[srcv.at[jl]], bufs[b], gsems[b]).wait()
            pltpu.sync_copy(bufs[b], acc_sh.at[dstv.at[jl]], add=True)

    plsc.subcore_barrier()

    @pl.loop(0, ZCOPIES)
    def _(k):
        off = sid * NSLICE + k * CHUNK
        pltpu.sync_copy(
            acc_sh.at[pl.ds(off, CHUNK)], outp_hbm.at[pl.ds(cid * NP + off, CHUNK)]
        )


@functools.cache
def _make_agg(W, NB, NSTAGE):
    return functools.partial(
        pl.kernel,
        out_type=jax.ShapeDtypeStruct((NC * NP, W), jnp.float32),
        mesh=plsc.VectorSubcoreMesh(core_axis_name="c", subcore_axis_name="s"),
        compiler_params=pltpu.CompilerParams(use_tc_tiling_on_sc=False),
        scratch_types=[
            pltpu.VMEM((NSTAGE, CHUNK), jnp.int32),
            pltpu.VMEM((NSTAGE, CHUNK), jnp.int32),
            *[pltpu.VMEM((CHUNK, W), jnp.float32) for _ in range(NB)],
            pltpu.VMEM_SHARED((NP, W), jnp.float32),
            pltpu.VMEM_SHARED((NP, W), jnp.float32),
            *[pltpu.SemaphoreType.DMA for _ in range(NB)],
        ],
    )(functools.partial(_agg_body, W, NB, NSTAGE))


def _tc1_body(degp_ref, x_ref, w1_ref, h1p_ref, dis_ref):
    dp = degp_ref[...]
    deg = (dp[:NP] + dp[NP:]).sum(axis=1, keepdims=True) * (1.0 / 16.0) + 1.0
    dis = lax.rsqrt(deg)
    h = jnp.dot(
        x_ref[...],
        w1_ref[...],
        preferred_element_type=jnp.float32,
        precision=lax.Precision.DEFAULT,
    )
    hp = h * dis[:N]
    zpad = jnp.zeros((NP - N, H // 2), jnp.float32)
    h1p_ref[pl.ds(0, N)] = hp[:, : H // 2]
    h1p_ref[pl.ds(N, NP - N)] = zpad
    h1p_ref[pl.ds(NP, N)] = hp[:, H // 2 :]
    h1p_ref[pl.ds(NP + N, NP - N)] = zpad
    dis_ref[...] = dis


def _tc2_body(acc_ref, h1p_ref, dis_ref, b1_ref, w2p_ref, gp_ref):
    dis = dis_ref[...]
    b1 = b1_ref[...]
    hw = H // 2
    hr_lo = jnp.maximum((acc_ref[:NP] + h1p_ref[:NP]) * dis + b1[None, :hw], 0.0)
    hr_hi = jnp.maximum((acc_ref[NP:] + h1p_ref[NP:]) * dis + b1[None, hw:], 0.0)
    g = jnp.dot(
        hr_lo,
        w2p_ref[:hw],
        preferred_element_type=jnp.float32,
        precision=lax.Precision.DEFAULT,
    ) + jnp.dot(
        hr_hi,
        w2p_ref[hw:],
        preferred_element_type=jnp.float32,
        precision=lax.Precision.DEFAULT,
    )
    gd = g * dis
    gp_ref[:NP] = gd[:, : CP // 2]
    gp_ref[NP:] = gd[:, CP // 2 :]


def _tc3_body(acc_ref, gp_ref, dis_ref, b2p_ref, feat_ref, logp_ref):
    s = jnp.concatenate(
        [
            acc_ref[pl.ds(0, N)] + gp_ref[pl.ds(0, N)],
            acc_ref[pl.ds(NP, N)] + gp_ref[pl.ds(NP, N)],
        ],
        axis=1,
    )
    out = s * dis_ref[pl.ds(0, N)] + b2p_ref[...][None, :]
    col = lax.broadcasted_iota(jnp.int32, (N, CP), 1)
    valid = col < C
    xm = jnp.where(valid, out, -jnp.inf)
    m = jnp.max(xm, axis=1, keepdims=True)
    e = jnp.where(valid, jnp.exp(out - m), 0.0)
    lse = m + jnp.log(jnp.sum(e, axis=1, keepdims=True))
    feat_ref[...] = out[:, :C]
    logp_ref[...] = (out - lse)[:, :C]


_tc1 = pl.pallas_call(
    _tc1_body,
    out_shape=[
        jax.ShapeDtypeStruct((NC * NP, H // 2), jnp.float32),
        jax.ShapeDtypeStruct((NP, 1), jnp.float32),
    ],
)

_tc2 = pl.pallas_call(
    _tc2_body,
    out_shape=jax.ShapeDtypeStruct((NC * NP, CP // 2), jnp.float32),
)

_tc3 = pl.pallas_call(
    _tc3_body,
    out_shape=[
        jax.ShapeDtypeStruct((N, C), jnp.float32),
        jax.ShapeDtypeStruct((N, C), jnp.float32),
    ],
)


def kernel(x, edge_index, W1, b1, W2, b2):
    src2 = jnp.full((EP,), N, jnp.int32).at[:E].set(edge_index[0]).reshape(
        ROWS_TOTAL, CHUNK
    )
    dst2 = jnp.full((EP,), N, jnp.int32).at[:E].set(edge_index[1]).reshape(
        ROWS_TOTAL, CHUNK
    )
    w2p = jnp.pad(W2, ((0, 0), (0, CP - C)))
    b2p = jnp.pad(b2, (0, CP - C))

    degp = _deg_kernel()(dst2)
    h1p, dis = _tc1(degp, x, W1)
    acc1 = _make_agg(H // 2, 4, 40)(h1p, src2, dst2)
    gp = _tc2(acc1, h1p, dis, b1, w2p)
    acc2 = _make_agg(CP // 2, 4, 40)(gp, src2, dst2)
    feat, logp = _tc3(acc2, gp, dis, b2p)
    return logp, feat
